# baseline (speedup 1.0000x reference)
import functools

import jax
import jax.numpy as jnp
from jax import lax
from jax.experimental import pallas as pl
from jax.experimental.pallas import tpu as pltpu
from jax.experimental.pallas import tpu_sc as plsc

N = 100000
D = 128
S = 128

BR = 512
SB = 128
NSB = BR // SB
NB_TC = 142
NTC = NB_TC * BR

NW = 32
CH = 864
T = 144
NT = CH // T
NV = D // 16
G = 16
NG = T // G


def _sc_partials(h_flat, ids):
    mesh = plsc.VectorSubcoreMesh(core_axis_name="c", subcore_axis_name="s")

    @functools.partial(
        pl.kernel,
        mesh=mesh,
        out_type=jax.ShapeDtypeStruct((NW * S * D,), jnp.float32),
        scratch_types=[
            pltpu.VMEM((CH,), jnp.int32),
            pltpu.VMEM((T * D,), jnp.float32),
            pltpu.VMEM((T * D,), jnp.float32),
            pltpu.VMEM((S * D,), jnp.float32),
            pltpu.SemaphoreType.DMA,
            pltpu.SemaphoreType.DMA,
        ],
    )
    def k(h_hbm, ids_hbm, out_hbm, ids_v, buf0, buf1, acc_v, sem0, sem1):
        wid = lax.axis_index("s") * 2 + lax.axis_index("c")
        base = NTC + ((wid * (N - NTC - CH)) // (NW - 1)) // 8 * 8
        base = pl.multiple_of(base, 8)
        bufs = (buf0, buf1)
        sems = (sem0, sem1)

        pltpu.sync_copy(ids_hbm.at[pl.ds(base, CH)], ids_v)

        neg = jnp.full((16,), -jnp.inf, dtype=jnp.float32)

        def init_blk(i, c):
            acc_v[pl.ds(i * 16, 16)] = neg
            return c

        lax.fori_loop(0, S * D // 16, init_blk, 0)

        def start_copy(t, b):
            pltpu.async_copy(
                h_hbm.at[pl.ds((base + t * T) * D, T * D)], bufs[b], sems[b]
            )

        def wait_copy(t, b):
            pltpu.make_async_copy(
                h_hbm.at[pl.ds((base + t * T) * D, T * D)], bufs[b], sems[b]
            ).wait()

        def process(t, b):
            @pl.when(t + 1 < NT)
            def _():
                start_copy(t + 1, 1 - b)

            wait_copy(t, b)
            buf = bufs[b]

            def group(j, c):
                row0 = j * G
                idv = ids_v[pl.ds(t * T + row0, G)]
                s0 = idv[0]
                uniform = s0 == idv[G - 1]

                @pl.when(uniform)
                def _():
                    for v in range(NV):
                        vals = [
                            buf[pl.ds((row0 + r) * D + v * 16, 16)]
                            for r in range(G)
                        ]
                        while len(vals) > 1:
                            vals = [
                                jnp.maximum(vals[i], vals[i + 1])
                                for i in range(0, len(vals) - 1, 2)
                            ] + ([vals[-1]] if len(vals) % 2 else [])
                        o = pl.ds(s0 * D + v * 16, 16)
                        acc_v[o] = jnp.maximum(acc_v[o], vals[0])

                @pl.when(jnp.logical_not(uniform))
                def _():
                    for r in range(G):
                        sid = idv[r]
                        for v in range(NV):
                            o = pl.ds(sid * D + v * 16, 16)
                            acc_v[o] = jnp.maximum(
                                acc_v[o], buf[pl.ds((row0 + r) * D + v * 16, 16)]
                            )

                return c

            lax.fori_loop(0, NG, group, 0)

        start_copy(0, 0)

        def pair(t, c):
            g = 2 * t
            process(g, 0)
            process(g + 1, 1)
            return c

        lax.fori_loop(0, NT // 2, pair, 0)

        pltpu.sync_copy(acc_v, out_hbm.at[pl.ds(wid * S * D, S * D)])

    return k(h_flat, ids)


def _tc_partial(h, ids3d):
    def body(ids_ref, idsv_ref, h_ref, o_ref, acc_ref):
        i = pl.program_id(0)

        @pl.when(i == 0)
        def _():
            acc_ref[...] = jnp.full((S, 1, D), -jnp.inf, dtype=jnp.float32)

        for sb in range(NSB):
            r0 = sb * SB
            s_first = ids_ref[0, 0, r0]
            s_last = ids_ref[0, 0, r0 + SB - 1]
            seg2 = idsv_ref[0, 0, pl.ds(r0, SB)].reshape(SB, 1)
            rows = h_ref[pl.ds(r0, SB), :]
            m1 = jnp.max(
                jnp.where(seg2 == s_first, rows, -jnp.inf), axis=0, keepdims=True
            )
            m2 = jnp.max(
                jnp.where(seg2 == s_last, rows, -jnp.inf), axis=0, keepdims=True
            )
            o1 = acc_ref[pl.ds(s_first, 1)]
            acc_ref[pl.ds(s_first, 1)] = jnp.maximum(o1, m1[None])
            o2 = acc_ref[pl.ds(s_last, 1)]
            acc_ref[pl.ds(s_last, 1)] = jnp.maximum(o2, m2[None])

            nmid = SB - jnp.sum((seg2 == s_first).astype(jnp.int32)) - jnp.sum(
                (seg2 == s_last).astype(jnp.int32)
            )

            @pl.when(nmid > 0)
            def _(r0=r0):
                def row(r, c):
                    sid = ids_ref[0, 0, r0 + r]
                    rv = h_ref[pl.ds(r0 + r, 1), :][None]
                    o = acc_ref[pl.ds(sid, 1)]
                    acc_ref[pl.ds(sid, 1)] = jnp.maximum(o, rv)
                    return c

                lax.fori_loop(0, SB, row, 0)

        @pl.when(i == NB_TC - 1)
        def _():
            o_ref[...] = acc_ref[:, 0, :]

    return pl.pallas_call(
        body,
        grid=(NB_TC,),
        in_specs=[
            pl.BlockSpec((1, 1, BR), lambda i: (i, 0, 0), memory_space=pltpu.SMEM),
            pl.BlockSpec((1, 1, BR), lambda i: (i, 0, 0)),
            pl.BlockSpec((BR, D), lambda i: (i, 0)),
        ],
        out_specs=pl.BlockSpec((S, D), lambda i: (0, 0)),
        out_shape=jax.ShapeDtypeStruct((S, D), jnp.float32),
        scratch_shapes=[pltpu.VMEM((S, 1, D), jnp.float32)],
    )(ids3d, ids3d, h)


def _merge(partials_sc, partial_tc):
    def body(p_ref, q_ref, o_ref):
        o_ref[...] = jnp.maximum(jnp.max(p_ref[...], axis=0), q_ref[...])

    return pl.pallas_call(
        body,
        out_shape=jax.ShapeDtypeStruct((S, D), jnp.float32),
    )(partials_sc, partial_tc)


def kernel(h, segment_ids):
    ids_tc = segment_ids[:NTC]
    partials_sc = _sc_partials(h.reshape(N * D), segment_ids)
    partial_tc = _tc_partial(h, ids_tc.reshape(NB_TC, 1, BR))
    return _merge(partials_sc.reshape(NW, S, D), partial_tc)


# speedup vs baseline: 2.1844x; 2.1844x over previous
import functools

import jax
import jax.numpy as jnp
from jax import lax
from jax.experimental import pallas as pl
from jax.experimental.pallas import tpu as pltpu
from jax.experimental.pallas import tpu_sc as plsc

N = 100000
D = 128
S = 128
NW = 32
CH = 3136
T = 224
NT = CH // T
NV = D // 16
G = 16
NG = T // G


def _sc_partials(h_flat, ids):
    mesh = plsc.VectorSubcoreMesh(core_axis_name="c", subcore_axis_name="s")

    @functools.partial(
        pl.kernel,
        mesh=mesh,
        out_type=jax.ShapeDtypeStruct((NW * S * D,), jnp.float32),
        scratch_types=[
            pltpu.VMEM((CH,), jnp.int32),
            pltpu.VMEM((T * D,), jnp.float32),
            pltpu.VMEM((T * D,), jnp.float32),
            pltpu.VMEM((S * D,), jnp.float32),
            pltpu.SemaphoreType.DMA,
            pltpu.SemaphoreType.DMA,
        ],
    )
    def k(h_hbm, ids_hbm, out_hbm, ids_v, buf0, buf1, acc_v, sem0, sem1):
        wid = lax.axis_index("s") * 2 + lax.axis_index("c")
        base = ((wid * (N - CH)) // (NW - 1)) // 8 * 8
        base = pl.multiple_of(base, 8)
        bufs = (buf0, buf1)
        sems = (sem0, sem1)

        def start_copy(t, b):
            pltpu.async_copy(
                h_hbm.at[pl.ds((base + t * T) * D, T * D)], bufs[b], sems[b]
            )

        def wait_copy(t, b):
            pltpu.make_async_copy(
                h_hbm.at[pl.ds((base + t * T) * D, T * D)], bufs[b], sems[b]
            ).wait()

        start_copy(0, 0)
        pltpu.sync_copy(ids_hbm.at[pl.ds(base, CH)], ids_v)

        neg = jnp.full((16,), -jnp.inf, dtype=jnp.float32)

        def init_blk(i, c):
            for u in range(8):
                acc_v[pl.ds(i * 128 + u * 16, 16)] = neg
            return c

        lax.fori_loop(0, S * D // 128, init_blk, 0)

        def process(t, b):
            @pl.when(t + 1 < NT)
            def _():
                start_copy(t + 1, 1 - b)

            wait_copy(t, b)
            buf = bufs[b]

            def tree_rmw(row0, nrows, sid):
                for v in range(NV):
                    vals = [
                        buf[pl.ds((row0 + r) * D + v * 16, 16)]
                        for r in range(nrows)
                    ]
                    while len(vals) > 1:
                        vals = [
                            jnp.maximum(vals[i], vals[i + 1])
                            for i in range(0, len(vals) - 1, 2)
                        ] + ([vals[-1]] if len(vals) % 2 else [])
                    o = pl.ds(sid * D + v * 16, 16)
                    acc_v[o] = jnp.maximum(acc_v[o], vals[0])

            def group16(row0, idv):
                s0 = idv[0]
                uniform = s0 == idv[G - 1]

                @pl.when(uniform)
                def _():
                    tree_rmw(row0, G, s0)

                @pl.when(jnp.logical_not(uniform))
                def _():
                    for r in range(G):
                        sid = idv[r]
                        for v in range(NV):
                            o = pl.ds(sid * D + v * 16, 16)
                            acc_v[o] = jnp.maximum(
                                acc_v[o], buf[pl.ds((row0 + r) * D + v * 16, 16)]
                            )

            def unit(j, c):
                row0 = j * 2 * G
                idv0 = ids_v[pl.ds(t * T + row0, G)]
                idv1 = ids_v[pl.ds(t * T + row0 + G, G)]
                s0 = idv0[0]
                uniform = s0 == idv1[G - 1]

                @pl.when(uniform)
                def _():
                    tree_rmw(row0, 2 * G, s0)

                @pl.when(jnp.logical_not(uniform))
                def _():
                    group16(row0, idv0)
                    group16(row0 + G, idv1)

                return c

            lax.fori_loop(0, NG // 2, unit, 0)

        def pair(t, c):
            g = 2 * t
            process(g, 0)
            process(g + 1, 1)
            return c

        lax.fori_loop(0, NT // 2, pair, 0)

        pltpu.sync_copy(acc_v, out_hbm.at[pl.ds(wid * S * D, S * D)])

    return k(h_flat, ids)


def _merge(partials):
    def body(p_ref, o_ref):
        o_ref[...] = jnp.max(p_ref[...], axis=0)

    return pl.pallas_call(
        body,
        out_shape=jax.ShapeDtypeStruct((S, D), jnp.float32),
    )(partials)


def kernel(h, segment_ids):
    partials = _sc_partials(h.reshape(N * D), segment_ids)
    return _merge(partials.reshape(NW, S, D))
